# baseline (speedup 1.0000x reference)
import functools

import jax
import jax.numpy as jnp
from jax import lax
from jax.experimental import pallas as pl
from jax.experimental.pallas import tpu as pltpu
from jax.experimental.pallas import tpu_sc as plsc

N = 16384
R = 32
K = 64
T = 32
B = 16
ALPHA = 0.9
V_TH = 1.0

NC = 2
NS = 16
NW = NC * NS
ROWS = N // NW
CPG = 128 // K
NCHUNK = ROWS // CPG
IDXROWS = N * K // 128

_MESH = plsc.VectorSubcoreMesh(core_axis_name="c", subcore_axis_name="s")
_CPARAMS = pltpu.CompilerParams(
    needs_layout_passes=False, use_tc_tiling_on_sc=False
)


def _wid():
    return lax.axis_index("s") * NC + lax.axis_index("c")


_W_NBUF = 4


@functools.partial(
    pl.kernel,
    out_type=jax.ShapeDtypeStruct((N, K), jnp.float32),
    mesh=_MESH,
    compiler_params=_CPARAMS,
    scratch_types=[
        pltpu.VMEM((NCHUNK, 128), jnp.int32),
        pltpu.VMEM((ROWS, R), jnp.float32),
        pltpu.VMEM((ROWS, K), jnp.float32),
        pltpu.VMEM((_W_NBUF, 128, R), jnp.float32),
    ] + [pltpu.SemaphoreType.DMA] * _W_NBUF,
)
def _w_fn(idx2_hbm, u_hbm, v_hbm, w_hbm, idx_vm, u_vm, w_vm, vbuf, *sems):
    wid = _wid()
    n0 = wid * ROWS
    c0 = wid * NCHUNK
    pltpu.sync_copy(idx2_hbm.at[pl.ds(c0, NCHUNK)], idx_vm)
    pltpu.sync_copy(u_hbm.at[pl.ds(n0, ROWS)], u_vm)
    for b in range(_W_NBUF):
        pltpu.async_copy(v_hbm.at[idx_vm.at[b]], vbuf.at[b], sems[b])

    @pl.loop(0, NCHUNK, step=_W_NBUF)
    def _chunk(j0):
        for b in range(_W_NBUF):
            j = j0 + b
            pltpu.make_async_copy(v_hbm.at[idx_vm.at[j]], vbuf.at[b], sems[b]).wait()
            lane = lax.iota(jnp.int32, 16)
            for a in range(CPG):
                nl = j * CPG + a
                u0 = u_vm[nl, pl.ds(0, 16)]
                u1 = u_vm[nl, pl.ds(16, 16)]
                for kb in range(K // 16):
                    acc = jnp.zeros((16,), jnp.float32)
                    for kk in range(16):
                        k = kb * 16 + kk
                        p = (vbuf[b, a * K + k, pl.ds(0, 16)] * u0
                             + vbuf[b, a * K + k, pl.ds(16, 16)] * u1)
                        dot = plsc.cumsum(p)[15]
                        acc = jnp.where(lane == kk, dot, acc)
                    w_vm[nl, pl.ds(kb * 16, 16)] = acc

            @pl.when(j + _W_NBUF < NCHUNK)
            def _refill():
                pltpu.async_copy(
                    v_hbm.at[idx_vm.at[j + _W_NBUF]], vbuf.at[b], sems[b]
                )

    pltpu.sync_copy(w_vm, w_hbm.at[pl.ds(n0, ROWS)])


@functools.partial(
    pl.kernel,
    out_type=(
        jax.ShapeDtypeStruct((N, B), jnp.float32),
        jax.ShapeDtypeStruct((N, B), jnp.float32),
    ),
    mesh=_MESH,
    compiler_params=_CPARAMS,
    scratch_types=[
        pltpu.VMEM((ROWS, B), jnp.float32),
        pltpu.VMEM((ROWS, B), jnp.float32),
        pltpu.VMEM((ROWS, B), jnp.float32),
    ],
)
def _t0_fn(x_hbm, v_hbm, a_hbm, x_vm, v_vm, a_vm):
    wid = _wid()
    n0 = wid * ROWS
    pltpu.sync_copy(x_hbm.at[pl.ds(n0, ROWS)], x_vm)

    @pl.loop(0, ROWS)
    def _row(i):
        xv = x_vm[i, :]
        spike = jnp.where(xv > V_TH, 1.0, 0.0).astype(jnp.float32)
        v_vm[i, :] = xv * (1.0 - spike)
        a_vm[i, :] = spike

    pltpu.sync_copy(v_vm, v_hbm.at[pl.ds(n0, ROWS)])
    pltpu.sync_copy(a_vm, a_hbm.at[pl.ds(n0, ROWS)])


_S_NBUF = 8
_S_HBM = 0


@functools.partial(
    pl.kernel,
    out_type=(
        jax.ShapeDtypeStruct((N, B), jnp.float32),
        jax.ShapeDtypeStruct((N, B), jnp.float32),
    ),
    mesh=_MESH,
    compiler_params=_CPARAMS,
    scratch_types=[
        pltpu.VMEM((NCHUNK, 128), jnp.int32),
        pltpu.VMEM((ROWS, K), jnp.float32),
        pltpu.VMEM((ROWS, B), jnp.float32),
        pltpu.VMEM((ROWS, B), jnp.float32),
        pltpu.VMEM((ROWS, B), jnp.float32),
        pltpu.VMEM((16,), jnp.float32),
        pltpu.VMEM((_S_NBUF, 128, B), jnp.float32),
        pltpu.VMEM_SHARED((N, B), jnp.float32),
    ] + [pltpu.SemaphoreType.DMA] * (_S_NBUF + 7),
)
def _step_fn(s_hbm, idx2_hbm, w_hbm, x_hbm, vin_hbm, ain_hbm, scale_hbm,
             vout_hbm, aout_hbm,
             idx_vm, w_vm, x_vm, v_vm, a_vm, sc_vm, gbuf, stab, *sems):
    psem = sems[_S_NBUF:]
    wid = _wid()
    sid = lax.axis_index("s")
    n0 = wid * ROWS
    c0 = wid * NCHUNK
    stage = [
        (s_hbm.at[pl.ds(sid * (N // NS), N // NS)],
         stab.at[pl.ds(sid * (N // NS), N // NS)]),
        (idx2_hbm.at[pl.ds(c0, NCHUNK)], idx_vm),
        (w_hbm.at[pl.ds(n0, ROWS)], w_vm),
        (x_hbm.at[pl.ds(n0, ROWS)], x_vm),
        (vin_hbm.at[pl.ds(n0, ROWS)], v_vm),
        (ain_hbm.at[pl.ds(n0, ROWS)], a_vm),
        (scale_hbm, sc_vm),
    ]
    for i, (src, dst) in enumerate(stage):
        pltpu.async_copy(src, dst, psem[i])
    for i, (src, dst) in enumerate(stage):
        pltpu.make_async_copy(src, dst, psem[i]).wait()
    plsc.subcore_barrier()

    def _src(b):
        return s_hbm if b < _S_HBM else stab

    for b in range(_S_NBUF):
        pltpu.async_copy(_src(b).at[idx_vm.at[b]], gbuf.at[b], sems[b])
    sv = sc_vm[...]

    @pl.loop(0, NCHUNK, step=_S_NBUF)
    def _chunk(j0):
        for b in range(_S_NBUF):
            j = j0 + b
            pltpu.make_async_copy(_src(b).at[idx_vm.at[j]], gbuf.at[b], sems[b]).wait()
            for a in range(CPG):
                nl = j * CPG + a
                syn = [jnp.zeros((16,), jnp.float32) for _ in range(4)]
                for kb in range(K // 16):
                    wrow = w_vm[nl, pl.ds(kb * 16, 16)]
                    for kk in range(16):
                        k = kb * 16 + kk
                        syn[kk % 4] = syn[kk % 4] + gbuf[b, a * K + k, :] * wrow[kk]
                syn = (syn[0] + syn[1]) + (syn[2] + syn[3])
                vv = ALPHA * v_vm[nl, :] + (x_vm[nl, :] + syn * sv)
                spike = jnp.where(vv > V_TH, 1.0, 0.0).astype(jnp.float32)
                v_vm[nl, :] = vv * (1.0 - spike)
                a_vm[nl, :] = a_vm[nl, :] + spike

            @pl.when(j + _S_NBUF < NCHUNK)
            def _refill():
                pltpu.async_copy(
                    _src(b).at[idx_vm.at[j + _S_NBUF]], gbuf.at[b], sems[b]
                )

    pltpu.async_copy(v_vm, vout_hbm.at[pl.ds(n0, ROWS)], psem[0])
    pltpu.async_copy(a_vm, aout_hbm.at[pl.ds(n0, ROWS)], psem[1])
    pltpu.make_async_copy(v_vm, vout_hbm.at[pl.ds(n0, ROWS)], psem[0]).wait()
    pltpu.make_async_copy(a_vm, aout_hbm.at[pl.ds(n0, ROWS)], psem[1]).wait()


def kernel(x, U, V, idx):
    xT = x.T.reshape(N, B)
    idx2 = idx.reshape(IDXROWS, 128)
    w = _w_fn(idx2, U, V)
    v0, a0 = _t0_fn(xT)

    def body(t, carry):
        v, a = carry
        scale = jnp.full((B,), 1.0, jnp.float32) / (t + 1).astype(jnp.float32)
        return _step_fn(a, idx2, w, xT, v, a, scale)

    _, a = lax.fori_loop(1, T, body, (v0, a0))
    return a.T.reshape(B, N)


# speedup vs baseline: 1.2946x; 1.2946x over previous
import functools

import jax
import jax.numpy as jnp
from jax import lax
from jax.experimental import pallas as pl
from jax.experimental.pallas import tpu as pltpu
from jax.experimental.pallas import tpu_sc as plsc

N = 16384
R = 32
K = 64
T = 32
B = 16
ALPHA = 0.9
V_TH = 1.0

NC = 2
NS = 16
NW = NC * NS
ROWS = N // NW
CPG = 128 // K
NCHUNK = ROWS // CPG
IDXROWS = N * K // 128

_MESH = plsc.VectorSubcoreMesh(core_axis_name="c", subcore_axis_name="s")
_CPARAMS = pltpu.CompilerParams(
    needs_layout_passes=False, use_tc_tiling_on_sc=False
)


def _wid():
    return lax.axis_index("s") * NC + lax.axis_index("c")


_W_NBUF = 4


@functools.partial(
    pl.kernel,
    out_type=jax.ShapeDtypeStruct((N, K), jnp.float32),
    mesh=_MESH,
    compiler_params=_CPARAMS,
    scratch_types=[
        pltpu.VMEM((NCHUNK, 128), jnp.int32),
        pltpu.VMEM((ROWS, R), jnp.float32),
        pltpu.VMEM((ROWS, K), jnp.float32),
        pltpu.VMEM((_W_NBUF, 128, R), jnp.float32),
    ] + [pltpu.SemaphoreType.DMA] * _W_NBUF,
)
def _w_fn(idx2_hbm, u_hbm, v_hbm, w_hbm, idx_vm, u_vm, w_vm, vbuf, *sems):
    wid = _wid()
    n0 = wid * ROWS
    c0 = wid * NCHUNK
    pltpu.sync_copy(idx2_hbm.at[pl.ds(c0, NCHUNK)], idx_vm)
    pltpu.sync_copy(u_hbm.at[pl.ds(n0, ROWS)], u_vm)
    for b in range(_W_NBUF):
        pltpu.async_copy(v_hbm.at[idx_vm.at[b]], vbuf.at[b], sems[b])

    @pl.loop(0, NCHUNK, step=_W_NBUF)
    def _chunk(j0):
        for b in range(_W_NBUF):
            j = j0 + b
            pltpu.make_async_copy(v_hbm.at[idx_vm.at[j]], vbuf.at[b], sems[b]).wait()
            lane = lax.iota(jnp.int32, 16)
            for a in range(CPG):
                nl = j * CPG + a
                u0 = u_vm[nl, pl.ds(0, 16)]
                u1 = u_vm[nl, pl.ds(16, 16)]
                for kb in range(K // 16):
                    acc = jnp.zeros((16,), jnp.float32)
                    for kk in range(16):
                        k = kb * 16 + kk
                        p = (vbuf[b, a * K + k, pl.ds(0, 16)] * u0
                             + vbuf[b, a * K + k, pl.ds(16, 16)] * u1)
                        dot = plsc.cumsum(p)[15]
                        acc = jnp.where(lane == kk, dot, acc)
                    w_vm[nl, pl.ds(kb * 16, 16)] = acc

            @pl.when(j + _W_NBUF < NCHUNK)
            def _refill():
                pltpu.async_copy(
                    v_hbm.at[idx_vm.at[j + _W_NBUF]], vbuf.at[b], sems[b]
                )

    pltpu.sync_copy(w_vm, w_hbm.at[pl.ds(n0, ROWS)])


@functools.partial(
    pl.kernel,
    out_type=(
        jax.ShapeDtypeStruct((N, B), jnp.float32),
        jax.ShapeDtypeStruct((N, B), jnp.float32),
    ),
    mesh=_MESH,
    compiler_params=_CPARAMS,
    scratch_types=[
        pltpu.VMEM((ROWS, B), jnp.float32),
        pltpu.VMEM((ROWS, B), jnp.float32),
        pltpu.VMEM((ROWS, B), jnp.float32),
    ],
)
def _t0_fn(x_hbm, v_hbm, a_hbm, x_vm, v_vm, a_vm):
    wid = _wid()
    n0 = wid * ROWS
    pltpu.sync_copy(x_hbm.at[pl.ds(n0, ROWS)], x_vm)

    @pl.loop(0, ROWS)
    def _row(i):
        xv = x_vm[i, :]
        spike = jnp.where(xv > V_TH, 1.0, 0.0).astype(jnp.float32)
        v_vm[i, :] = xv * (1.0 - spike)
        a_vm[i, :] = spike

    pltpu.sync_copy(v_vm, v_hbm.at[pl.ds(n0, ROWS)])
    pltpu.sync_copy(a_vm, a_hbm.at[pl.ds(n0, ROWS)])


_S_NBUF = 8
_S_HBM = 0


@functools.partial(
    pl.kernel,
    out_type=(
        jax.ShapeDtypeStruct((N, B), jnp.float32),
        jax.ShapeDtypeStruct((N, B), jnp.float32),
    ),
    mesh=_MESH,
    compiler_params=_CPARAMS,
    scratch_types=[
        pltpu.VMEM((NCHUNK, 128), jnp.int32),
        pltpu.VMEM((ROWS, K), jnp.float32),
        pltpu.VMEM((ROWS, B), jnp.float32),
        pltpu.VMEM((ROWS, B), jnp.float32),
        pltpu.VMEM((ROWS, B), jnp.float32),
        pltpu.VMEM((16,), jnp.float32),
        pltpu.VMEM((_S_NBUF, 128, B), jnp.float32),
        pltpu.VMEM_SHARED((N, B), jnp.float32),
    ] + [pltpu.SemaphoreType.DMA] * (_S_NBUF + 7),
)
def _step_fn(s_hbm, idx2_hbm, w_hbm, x_hbm, vin_hbm, ain_hbm, scale_hbm,
             vout_hbm, aout_hbm,
             idx_vm, w_vm, x_vm, v_vm, a_vm, sc_vm, gbuf, stab, *sems):
    psem = sems[_S_NBUF:]
    wid = _wid()
    sid = lax.axis_index("s")
    n0 = wid * ROWS
    c0 = wid * NCHUNK
    stage = [
        (s_hbm.at[pl.ds(sid * (N // NS), N // NS)],
         stab.at[pl.ds(sid * (N // NS), N // NS)]),
        (idx2_hbm.at[pl.ds(c0, NCHUNK)], idx_vm),
        (w_hbm.at[pl.ds(n0, ROWS)], w_vm),
        (x_hbm.at[pl.ds(n0, ROWS)], x_vm),
        (vin_hbm.at[pl.ds(n0, ROWS)], v_vm),
        (ain_hbm.at[pl.ds(n0, ROWS)], a_vm),
        (scale_hbm, sc_vm),
    ]
    for i, (src, dst) in enumerate(stage):
        pltpu.async_copy(src, dst, psem[i])
    for i, (src, dst) in enumerate(stage):
        pltpu.make_async_copy(src, dst, psem[i]).wait()
    plsc.subcore_barrier()

    def _src(b):
        return s_hbm if b < _S_HBM else stab

    for b in range(_S_NBUF):
        pltpu.async_copy(_src(b).at[idx_vm.at[b]], gbuf.at[b], sems[b])
    sv = sc_vm[...]

    @pl.loop(0, NCHUNK, step=_S_NBUF)
    def _chunk(j0):
        for b in range(_S_NBUF):
            j = j0 + b
            pltpu.make_async_copy(_src(b).at[idx_vm.at[j]], gbuf.at[b], sems[b]).wait()
            for a in range(CPG):
                nl = j * CPG + a
                syn = [jnp.zeros((16,), jnp.float32) for _ in range(4)]
                for kb in range(K // 16):
                    wrow = w_vm[nl, pl.ds(kb * 16, 16)]
                    for kk in range(16):
                        k = kb * 16 + kk
                        syn[kk % 4] = syn[kk % 4] + gbuf[b, a * K + k, :] * wrow[kk]
                syn = (syn[0] + syn[1]) + (syn[2] + syn[3])
                vv = ALPHA * v_vm[nl, :] + (x_vm[nl, :] + syn * sv)
                spike = jnp.where(vv > V_TH, 1.0, 0.0).astype(jnp.float32)
                v_vm[nl, :] = vv * (1.0 - spike)
                a_vm[nl, :] = a_vm[nl, :] + spike

            @pl.when(j + _S_NBUF < NCHUNK)
            def _refill():
                pltpu.async_copy(
                    _src(b).at[idx_vm.at[j + _S_NBUF]], gbuf.at[b], sems[b]
                )

    pltpu.async_copy(v_vm, vout_hbm.at[pl.ds(n0, ROWS)], psem[0])
    pltpu.async_copy(a_vm, aout_hbm.at[pl.ds(n0, ROWS)], psem[1])
    pltpu.make_async_copy(v_vm, vout_hbm.at[pl.ds(n0, ROWS)], psem[0]).wait()
    pltpu.make_async_copy(a_vm, aout_hbm.at[pl.ds(n0, ROWS)], psem[1]).wait()


def kernel(x, U, V, idx):
    xT = x.T.reshape(N, B)
    idx2 = idx.reshape(IDXROWS, 128)
    w = _w_fn(idx2, U, V)
    v, a = _t0_fn(xT)
    for t in range(1, T):
        scale = jnp.full((B,), 1.0 / (t + 1), jnp.float32)
        v, a = _step_fn(a, idx2, w, xT, v, a, scale)
    return a.T.reshape(B, N)


# speedup vs baseline: 1.2950x; 1.0003x over previous
import functools

import jax
import jax.numpy as jnp
from jax import lax
from jax.experimental import pallas as pl
from jax.experimental.pallas import tpu as pltpu
from jax.experimental.pallas import tpu_sc as plsc

N = 16384
R = 32
K = 64
T = 32
B = 16
ALPHA = 0.9
V_TH = 1.0

NC = 2
NS = 16
NW = NC * NS
ROWS = N // NW
CPG = 128 // K
NCHUNK = ROWS // CPG
IDXROWS = N * K // 128

_MESH = plsc.VectorSubcoreMesh(core_axis_name="c", subcore_axis_name="s")
_CPARAMS = pltpu.CompilerParams(
    needs_layout_passes=False, use_tc_tiling_on_sc=False
)


def _wid():
    return lax.axis_index("s") * NC + lax.axis_index("c")


_W_NBUF = 4


@functools.partial(
    pl.kernel,
    out_type=jax.ShapeDtypeStruct((N, K), jnp.float32),
    mesh=_MESH,
    compiler_params=_CPARAMS,
    scratch_types=[
        pltpu.VMEM((NCHUNK, 128), jnp.int32),
        pltpu.VMEM((ROWS, R), jnp.float32),
        pltpu.VMEM((ROWS, K), jnp.float32),
        pltpu.VMEM((_W_NBUF, 128, R), jnp.float32),
    ] + [pltpu.SemaphoreType.DMA] * _W_NBUF,
)
def _w_fn(idx2_hbm, u_hbm, v_hbm, w_hbm, idx_vm, u_vm, w_vm, vbuf, *sems):
    wid = _wid()
    n0 = wid * ROWS
    c0 = wid * NCHUNK
    pltpu.sync_copy(idx2_hbm.at[pl.ds(c0, NCHUNK)], idx_vm)
    pltpu.sync_copy(u_hbm.at[pl.ds(n0, ROWS)], u_vm)
    for b in range(_W_NBUF):
        pltpu.async_copy(v_hbm.at[idx_vm.at[b]], vbuf.at[b], sems[b])

    @pl.loop(0, NCHUNK, step=_W_NBUF)
    def _chunk(j0):
        for b in range(_W_NBUF):
            j = j0 + b
            pltpu.make_async_copy(v_hbm.at[idx_vm.at[j]], vbuf.at[b], sems[b]).wait()
            lane = lax.iota(jnp.int32, 16)
            for a in range(CPG):
                nl = j * CPG + a
                u0 = u_vm[nl, pl.ds(0, 16)]
                u1 = u_vm[nl, pl.ds(16, 16)]
                for kb in range(K // 16):
                    acc = jnp.zeros((16,), jnp.float32)
                    for kk in range(16):
                        k = kb * 16 + kk
                        p = (vbuf[b, a * K + k, pl.ds(0, 16)] * u0
                             + vbuf[b, a * K + k, pl.ds(16, 16)] * u1)
                        dot = plsc.cumsum(p)[15]
                        acc = jnp.where(lane == kk, dot, acc)
                    w_vm[nl, pl.ds(kb * 16, 16)] = acc

            @pl.when(j + _W_NBUF < NCHUNK)
            def _refill():
                pltpu.async_copy(
                    v_hbm.at[idx_vm.at[j + _W_NBUF]], vbuf.at[b], sems[b]
                )

    pltpu.sync_copy(w_vm, w_hbm.at[pl.ds(n0, ROWS)])


@functools.partial(
    pl.kernel,
    out_type=(
        jax.ShapeDtypeStruct((N, B), jnp.float32),
        jax.ShapeDtypeStruct((N, B), jnp.float32),
    ),
    mesh=_MESH,
    compiler_params=_CPARAMS,
    scratch_types=[
        pltpu.VMEM((ROWS, B), jnp.float32),
        pltpu.VMEM((ROWS, B), jnp.float32),
        pltpu.VMEM((ROWS, B), jnp.float32),
    ],
)
def _t0_fn(x_hbm, v_hbm, a_hbm, x_vm, v_vm, a_vm):
    wid = _wid()
    n0 = wid * ROWS
    pltpu.sync_copy(x_hbm.at[pl.ds(n0, ROWS)], x_vm)

    @pl.loop(0, ROWS)
    def _row(i):
        xv = x_vm[i, :]
        spike = jnp.where(xv > V_TH, 1.0, 0.0).astype(jnp.float32)
        v_vm[i, :] = xv * (1.0 - spike)
        a_vm[i, :] = spike

    pltpu.sync_copy(v_vm, v_hbm.at[pl.ds(n0, ROWS)])
    pltpu.sync_copy(a_vm, a_hbm.at[pl.ds(n0, ROWS)])


_S_NBUF = 8


@functools.partial(
    pl.kernel,
    out_type=(
        jax.ShapeDtypeStruct((N, B), jnp.float32),
        jax.ShapeDtypeStruct((N, B), jnp.float32),
    ),
    mesh=_MESH,
    compiler_params=_CPARAMS,
    scratch_types=[
        pltpu.VMEM((NCHUNK, 128), jnp.int32),
        pltpu.VMEM((ROWS, K), jnp.float32),
        pltpu.VMEM((ROWS, B), jnp.float32),
        pltpu.VMEM((ROWS, B), jnp.float32),
        pltpu.VMEM((ROWS, B), jnp.float32),
        pltpu.VMEM((16,), jnp.float32),
        pltpu.VMEM((_S_NBUF, 128, B), jnp.float32),
        pltpu.VMEM_SHARED((N, B), jnp.float32),
    ] + [pltpu.SemaphoreType.DMA] * (_S_NBUF + 7),
)
def _step_fn(s_hbm, idx2_hbm, w_hbm, x_hbm, vin_hbm, ain_hbm, scale_hbm,
             vout_hbm, aout_hbm,
             idx_vm, w_vm, x_vm, v_vm, a_vm, sc_vm, gbuf, stab, *sems):
    psem = sems[_S_NBUF:]
    wid = _wid()
    sid = lax.axis_index("s")
    n0 = wid * ROWS
    c0 = wid * NCHUNK
    stage = [
        (s_hbm.at[pl.ds(sid * (N // NS), N // NS)],
         stab.at[pl.ds(sid * (N // NS), N // NS)]),
        (idx2_hbm.at[pl.ds(c0, NCHUNK)], idx_vm),
        (w_hbm.at[pl.ds(n0, ROWS)], w_vm),
        (x_hbm.at[pl.ds(n0, ROWS)], x_vm),
        (vin_hbm.at[pl.ds(n0, ROWS)], v_vm),
        (ain_hbm.at[pl.ds(n0, ROWS)], a_vm),
        (scale_hbm, sc_vm),
    ]
    for i, (src, dst) in enumerate(stage):
        pltpu.async_copy(src, dst, psem[i])
    for i, (src, dst) in enumerate(stage):
        pltpu.make_async_copy(src, dst, psem[i]).wait()
    plsc.subcore_barrier()

    for b in range(_S_NBUF):
        pltpu.async_copy(stab.at[idx_vm.at[b]], gbuf.at[b], sems[b])
    sv = sc_vm[...]

    @pl.loop(0, NCHUNK, step=_S_NBUF)
    def _chunk(j0):
        for b in range(_S_NBUF):
            j = j0 + b
            pltpu.make_async_copy(stab.at[idx_vm.at[j]], gbuf.at[b], sems[b]).wait()
            for a in range(CPG):
                nl = j * CPG + a
                syn = [jnp.zeros((16,), jnp.float32) for _ in range(4)]
                for kb in range(K // 16):
                    wrow = w_vm[nl, pl.ds(kb * 16, 16)]
                    for kk in range(16):
                        k = kb * 16 + kk
                        syn[kk % 4] = syn[kk % 4] + gbuf[b, a * K + k, :] * wrow[kk]
                syn = (syn[0] + syn[1]) + (syn[2] + syn[3])
                vv = ALPHA * v_vm[nl, :] + (x_vm[nl, :] + syn * sv)
                spike = jnp.where(vv > V_TH, 1.0, 0.0).astype(jnp.float32)
                v_vm[nl, :] = vv * (1.0 - spike)
                a_vm[nl, :] = a_vm[nl, :] + spike

            @pl.when(j + _S_NBUF < NCHUNK)
            def _refill():
                pltpu.async_copy(
                    stab.at[idx_vm.at[j + _S_NBUF]], gbuf.at[b], sems[b]
                )

    pltpu.async_copy(v_vm, vout_hbm.at[pl.ds(n0, ROWS)], psem[0])
    pltpu.async_copy(a_vm, aout_hbm.at[pl.ds(n0, ROWS)], psem[1])
    pltpu.make_async_copy(v_vm, vout_hbm.at[pl.ds(n0, ROWS)], psem[0]).wait()
    pltpu.make_async_copy(a_vm, aout_hbm.at[pl.ds(n0, ROWS)], psem[1]).wait()


def kernel(x, U, V, idx):
    xT = x.T.reshape(N, B)
    idx2 = idx.reshape(IDXROWS, 128)
    w = _w_fn(idx2, U, V)
    v, a = _t0_fn(xT)
    for t in range(1, T):
        scale = jnp.full((B,), 1.0 / (t + 1), jnp.float32)
        v, a = _step_fn(a, idx2, w, xT, v, a, scale)
    return a.T.reshape(B, N)
